# baseline (speedup 1.0000x reference)
import jax
import jax.numpy as jnp
from jax import lax
from jax.experimental import pallas as pl
from jax.experimental.pallas import tpu as pltpu
from jax.experimental.pallas import tpu_sc as plsc

_LANES = 16
_VALW = 128
_C = 1792
_GCAP = 768
_PIECE = 128


def _make_fused_kernel(M, D, B, NC, NS):
    NW = NC * NS
    maligned = M // 128 * 128
    wcols = maligned // NW // 128 * 128
    extra_start = wcols * NW
    extra_w = maligned - extra_start
    nfull = wcols // _C
    rem = wcols - nfull * _C
    mesh = plsc.VectorSubcoreMesh(core_axis_name="c", subcore_axis_name="s")

    def body(memt_hbm, idx_hbm, valp_hbm, lo_hbm, up_hbm, out_hbm,
             slab_v, idx_v, gcol_v, gb_v, stage_v, dest_v,
             lo_v, up_v, sem):
        wid = lax.axis_index("s") * NC + lax.axis_index("c")
        base = wid * wcols
        wend = jnp.where(wid == NW - 1, maligned, base + wcols)

        pltpu.sync_copy(lo_hbm, lo_v)
        pltpu.sync_copy(up_hbm, up_v)

        lanes = lax.iota(jnp.int32, _LANES)
        zeros = jnp.zeros((_LANES,), jnp.int32)
        for z in range(_GCAP // _LANES):
            gb_v[pl.ds(z * _LANES, _LANES)] = zeros

        def popcount(m):
            return lax.reduce_max(plsc.all_reduce_population_count(m), (0,))

        half = B // 2
        gcount = 0
        for h in range(2):
            pltpu.sync_copy(idx_hbm.at[pl.ds(h * half, half)], idx_v)

            def scan_step(t, gc, h=h):
                v = idx_v[pl.ds(t * _LANES, _LANES)]
                m = jnp.logical_and(v >= base, v < wend)
                plsc.store_compressed(gcol_v.at[pl.ds(gc, _LANES)], v, mask=m)
                plsc.store_compressed(gb_v.at[pl.ds(gc, _LANES)],
                                      h * half + t * _LANES + lanes, mask=m)
                return jnp.minimum(gc + popcount(m), _GCAP)

            gcount = lax.fori_loop(0, half // _LANES, scan_step, gcount,
                                   unroll=4)
        ngrp = (gcount + _LANES - 1) // _LANES

        npiece = _GCAP // _PIECE

        def fire(p):
            return pltpu.async_copy(
                valp_hbm.at[gb_v.at[pl.ds(p * _PIECE, _PIECE)]],
                stage_v.at[pl.ds((p % 2) * _PIECE, _PIECE)], sem)

        cps = {0: fire(0), 1: fire(1)}
        for p in range(npiece):
            cps.pop(p).wait()
            rows = jnp.clip(gcount - p * _PIECE, 0, _PIECE)
            ngrp_p = (rows + _LANES - 1) // _LANES

            @plsc.parallel_loop(0, ngrp_p, unroll=1)
            def tp(j, p=p):
                ridx = (p % 2) * _PIECE + j * _LANES + lanes
                for c in range(D):
                    csplat = jnp.zeros((_LANES,), jnp.int32) + c
                    locs = plsc.load_gather(lo_v, [csplat])
                    upcs = plsc.load_gather(up_v, [csplat])
                    g = plsc.load_gather(stage_v, [ridx, csplat])
                    d = jnp.where(g >= upcs, locs,
                                  jnp.where(g <= locs, upcs, g))
                    dest_v[c, pl.ds(p * _PIECE + j * _LANES, _LANES)] = d

            if p + 2 < npiece:
                cps[p + 2] = fire(p + 2)

        def process_chunk(cs, width):
            cs = pl.multiple_of(cs, 128)
            pltpu.sync_copy(memt_hbm.at[:, pl.ds(cs, width)],
                            slab_v.at[:, pl.ds(0, width)])

            def sweep(t, _):
                cols = gcol_v[pl.ds(t * _LANES, _LANES)]
                slot = t * _LANES + lanes
                m = jnp.logical_and(slot < gcount,
                                    jnp.logical_and(cols >= cs,
                                                    cols < cs + width))
                rloc = cols - cs
                for c in range(D):
                    x = dest_v[c, pl.ds(t * _LANES, _LANES)]
                    plsc.store_scatter(
                        slab_v,
                        [jnp.zeros((_LANES,), jnp.int32) + c, rloc],
                        x, mask=m)
                return 0

            lax.fori_loop(0, ngrp, sweep, 0)

            pltpu.sync_copy(slab_v.at[:, pl.ds(0, width)],
                            out_hbm.at[:, pl.ds(cs, width)])

        def chunk_step(ch, _):
            process_chunk(base + ch * _C, _C)
            return 0

        lax.fori_loop(0, nfull, chunk_step, 0)
        if rem:
            process_chunk(base + nfull * _C, rem)

        if extra_w:
            @pl.when(wid == NW - 1)
            def _():
                process_chunk(jnp.int32(extra_start), extra_w)

    return pl.kernel(
        body,
        out_type=jax.ShapeDtypeStruct((D, M), jnp.float32),
        mesh=mesh,
        scratch_types=[
            pltpu.VMEM((D, _C), jnp.float32),
            pltpu.VMEM((B // 2,), jnp.int32),
            pltpu.VMEM((_GCAP + _LANES,), jnp.int32),
            pltpu.VMEM((_GCAP + _LANES,), jnp.int32),
            pltpu.VMEM((2 * _PIECE, _VALW), jnp.float32),
            pltpu.VMEM((D, _GCAP), jnp.float32),
            pltpu.VMEM((D,), jnp.float32),
            pltpu.VMEM((D,), jnp.float32),
            pltpu.SemaphoreType.DMA,
        ],
        compiler_params=pltpu.CompilerParams(use_tc_tiling_on_sc=True,
                                             needs_layout_passes=False),
    )


def _swap_clamp(v, lower, upper):
    big = v >= upper[None, :]
    small = v <= lower[None, :]
    return jnp.where(big, lower[None, :], jnp.where(small, upper[None, :], v))


def kernel(mem, idx, val, lower, upper):
    M, D = mem.shape
    B = idx.shape[0]
    NC, NS = 2, 16

    valp = jnp.pad(val, ((0, 0), (0, _VALW - D)))
    fused = _make_fused_kernel(M, D, B, NC, NS)
    outt = fused(mem.T, idx, valp, lower, upper)

    maligned = M // 128 * 128
    ntail = M - maligned
    if ntail:
        cols = maligned + jnp.arange(ntail)
        eq = idx[:, None] == cols[None, :]
        winner = jnp.max(jnp.where(eq, jnp.arange(B)[:, None], -1), axis=0)
        wrow = _swap_clamp(val[jnp.maximum(winner, 0)], lower, upper)
        tail = jnp.where((winner >= 0)[None, :], wrow.T,
                         lax.dynamic_slice(mem.T, (0, maligned), (D, ntail)))
        outt = lax.dynamic_update_slice(outt, tail, (0, maligned))
    return outt.T


# speedup vs baseline: 1.0018x; 1.0018x over previous
import jax
import jax.numpy as jnp
from jax import lax
from jax.experimental import pallas as pl
from jax.experimental.pallas import tpu as pltpu
from jax.experimental.pallas import tpu_sc as plsc

_LANES = 16
_VALW = 128
_C = 1792
_GCAP = 768
_PIECE = 128


def _make_fused_kernel(M, D, B, NC, NS):
    NW = NC * NS
    maligned = M // 128 * 128
    wcols = maligned // NW // 128 * 128
    extra_start = wcols * NW
    extra_w = maligned - extra_start
    nfull = wcols // _C
    rem = wcols - nfull * _C
    mesh = plsc.VectorSubcoreMesh(core_axis_name="c", subcore_axis_name="s")

    def body(memt_hbm, idx_hbm, valp_hbm, lo_hbm, up_hbm, out_hbm,
             slab_v, idx_v, gcol_v, gb_v, stage_v, dest_v,
             lo_v, up_v, sem):
        wid = lax.axis_index("s") * NC + lax.axis_index("c")
        base = wid * wcols
        wend = jnp.where(wid == NW - 1, maligned, base + wcols)

        pltpu.sync_copy(lo_hbm, lo_v)
        pltpu.sync_copy(up_hbm, up_v)

        lanes = lax.iota(jnp.int32, _LANES)
        zeros = jnp.zeros((_LANES,), jnp.int32)
        for z in range(_GCAP // _LANES):
            gb_v[pl.ds(z * _LANES, _LANES)] = zeros

        def popcount(m):
            return lax.reduce_max(plsc.all_reduce_population_count(m), (0,))

        half = B // 2
        gcount = 0
        for h in range(2):
            pltpu.sync_copy(idx_hbm.at[pl.ds(h * half, half)], idx_v)

            def scan_step(t, gc, h=h):
                v = idx_v[pl.ds(t * _LANES, _LANES)]
                m = jnp.logical_and(v >= base, v < wend)
                plsc.store_compressed(gcol_v.at[pl.ds(gc, _LANES)], v, mask=m)
                plsc.store_compressed(gb_v.at[pl.ds(gc, _LANES)],
                                      h * half + t * _LANES + lanes, mask=m)
                return jnp.minimum(gc + popcount(m), _GCAP)

            gcount = lax.fori_loop(0, half // _LANES, scan_step, gcount,
                                   unroll=8)
        ngrp = (gcount + _LANES - 1) // _LANES

        npiece = _GCAP // _PIECE

        def fire(p):
            pltpu.async_copy(
                valp_hbm.at[gb_v.at[pl.ds(p * _PIECE, _PIECE)]],
                stage_v.at[pl.ds((p % 2) * _PIECE, _PIECE)], sem)

        fire(0)
        fire(1)

        def piece(p, _):
            pltpu.make_async_copy(
                valp_hbm.at[gb_v.at[pl.ds(p * _PIECE, _PIECE)]],
                stage_v.at[pl.ds((p % 2) * _PIECE, _PIECE)], sem).wait()
            rows = jnp.clip(gcount - p * _PIECE, 0, _PIECE)
            ngrp_p = (rows + _LANES - 1) // _LANES

            @plsc.parallel_loop(0, ngrp_p, unroll=2)
            def tp(j):
                ridx = (p % 2) * _PIECE + j * _LANES + lanes
                for c in range(D):
                    csplat = jnp.zeros((_LANES,), jnp.int32) + c
                    locs = plsc.load_gather(lo_v, [csplat])
                    upcs = plsc.load_gather(up_v, [csplat])
                    g = plsc.load_gather(stage_v, [ridx, csplat])
                    d = jnp.where(g >= upcs, locs,
                                  jnp.where(g <= locs, upcs, g))
                    dest_v[c, pl.ds(p * _PIECE + j * _LANES, _LANES)] = d

            @pl.when(p + 2 < npiece)
            def _():
                fire(p + 2)
            return 0

        lax.fori_loop(0, npiece, piece, 0)

        def process_chunk(cs, width):
            cs = pl.multiple_of(cs, 128)
            pltpu.sync_copy(memt_hbm.at[:, pl.ds(cs, width)],
                            slab_v.at[:, pl.ds(0, width)])

            def sweep(t, _):
                cols = gcol_v[pl.ds(t * _LANES, _LANES)]
                slot = t * _LANES + lanes
                m = jnp.logical_and(slot < gcount,
                                    jnp.logical_and(cols >= cs,
                                                    cols < cs + width))
                rloc = cols - cs
                for c in range(D):
                    x = dest_v[c, pl.ds(t * _LANES, _LANES)]
                    plsc.store_scatter(
                        slab_v,
                        [jnp.zeros((_LANES,), jnp.int32) + c, rloc],
                        x, mask=m)
                return 0

            lax.fori_loop(0, ngrp, sweep, 0)

            pltpu.sync_copy(slab_v.at[:, pl.ds(0, width)],
                            out_hbm.at[:, pl.ds(cs, width)])

        def chunk_step(ch, _):
            process_chunk(base + ch * _C, _C)
            return 0

        lax.fori_loop(0, nfull, chunk_step, 0)
        if rem:
            process_chunk(base + nfull * _C, rem)

        if extra_w:
            @pl.when(wid == NW - 1)
            def _():
                process_chunk(jnp.int32(extra_start), extra_w)

    return pl.kernel(
        body,
        out_type=jax.ShapeDtypeStruct((D, M), jnp.float32),
        mesh=mesh,
        scratch_types=[
            pltpu.VMEM((D, _C), jnp.float32),
            pltpu.VMEM((B // 2,), jnp.int32),
            pltpu.VMEM((_GCAP + _LANES,), jnp.int32),
            pltpu.VMEM((_GCAP + _LANES,), jnp.int32),
            pltpu.VMEM((2 * _PIECE, _VALW), jnp.float32),
            pltpu.VMEM((D, _GCAP), jnp.float32),
            pltpu.VMEM((D,), jnp.float32),
            pltpu.VMEM((D,), jnp.float32),
            pltpu.SemaphoreType.DMA,
        ],
        compiler_params=pltpu.CompilerParams(use_tc_tiling_on_sc=True,
                                             needs_layout_passes=False),
    )


def _swap_clamp(v, lower, upper):
    big = v >= upper[None, :]
    small = v <= lower[None, :]
    return jnp.where(big, lower[None, :], jnp.where(small, upper[None, :], v))


def kernel(mem, idx, val, lower, upper):
    M, D = mem.shape
    B = idx.shape[0]
    NC, NS = 2, 16

    valp = jnp.pad(val, ((0, 0), (0, _VALW - D)))
    fused = _make_fused_kernel(M, D, B, NC, NS)
    outt = fused(mem.T, idx, valp, lower, upper)

    maligned = M // 128 * 128
    ntail = M - maligned
    if ntail:
        cols = maligned + jnp.arange(ntail)
        eq = idx[:, None] == cols[None, :]
        winner = jnp.max(jnp.where(eq, jnp.arange(B)[:, None], -1), axis=0)
        wrow = _swap_clamp(val[jnp.maximum(winner, 0)], lower, upper)
        tail = jnp.where((winner >= 0)[None, :], wrow.T,
                         lax.dynamic_slice(mem.T, (0, maligned), (D, ntail)))
        outt = lax.dynamic_update_slice(outt, tail, (0, maligned))
    return outt.T


# speedup vs baseline: 1.5500x; 1.5472x over previous
import jax
import jax.numpy as jnp
from jax import lax
from jax.experimental import pallas as pl
from jax.experimental.pallas import tpu as pltpu
from jax.experimental.pallas import tpu_sc as plsc

_LANES = 16
_VALW = 128
_C = 1792
_GCAP = 1024
_PIECE = 128


def _make_fused_kernel(M, D, B, NC, NS):
    NW = NC * NS
    maligned = M // 128 * 128
    wcols = maligned // NW // 128 * 128
    extra_start = wcols * NW
    extra_w = maligned - extra_start
    nfull = wcols // _C
    rem = wcols - nfull * _C
    mesh = plsc.VectorSubcoreMesh(core_axis_name="c", subcore_axis_name="s")

    def body(memt_hbm, idx_hbm, valp_hbm, lo_hbm, up_hbm, out_hbm,
             slab_v, idx_v, gcol_v, gb_v, stage_v, dest_v,
             lo_v, up_v, sem):
        wid = lax.axis_index("s") * NC + lax.axis_index("c")
        base = wid * wcols
        wend = jnp.where(wid == NW - 1, maligned, base + wcols)

        pltpu.sync_copy(idx_hbm, idx_v)
        pltpu.sync_copy(lo_hbm, lo_v)
        pltpu.sync_copy(up_hbm, up_v)

        lanes = lax.iota(jnp.int32, _LANES)
        zeros = jnp.zeros((_LANES,), jnp.int32)
        for z in range(_GCAP // _LANES):
            gb_v[pl.ds(z * _LANES, _LANES)] = zeros

        def popcount(m):
            return lax.reduce_max(plsc.all_reduce_population_count(m), (0,))

        def scan_step(t, gc):
            v = idx_v[pl.ds(t * _LANES, _LANES)]
            m = jnp.logical_and(v >= base, v < wend)
            plsc.store_compressed(gcol_v.at[pl.ds(gc, _LANES)], v, mask=m)
            plsc.store_compressed(gb_v.at[pl.ds(gc, _LANES)],
                                  t * _LANES + lanes, mask=m)
            return jnp.minimum(gc + popcount(m), _GCAP)

        gcount = lax.fori_loop(0, B // _LANES, scan_step, 0, unroll=8)
        ngrp = (gcount + _LANES - 1) // _LANES

        def piece(p, _):
            pltpu.async_copy(valp_hbm.at[gb_v.at[pl.ds(p * _PIECE, _PIECE)]],
                             stage_v, sem).wait()
            rows = jnp.minimum(gcount - p * _PIECE, _PIECE)
            ngrp_p = (rows + _LANES - 1) // _LANES

            @plsc.parallel_loop(0, ngrp_p, unroll=2)
            def tp(j):
                ridx = j * _LANES + lanes
                for c in range(D):
                    csplat = jnp.zeros((_LANES,), jnp.int32) + c
                    locs = plsc.load_gather(lo_v, [csplat])
                    upcs = plsc.load_gather(up_v, [csplat])
                    g = plsc.load_gather(stage_v, [ridx, csplat])
                    d = jnp.where(g >= upcs, locs,
                                  jnp.where(g <= locs, upcs, g))
                    dest_v[c, pl.ds(p * _PIECE + j * _LANES, _LANES)] = d
            return 0

        npiece = (gcount + _PIECE - 1) // _PIECE
        lax.fori_loop(0, npiece, piece, 0)

        def process_chunk(cs, width):
            cs = pl.multiple_of(cs, 128)
            pltpu.sync_copy(memt_hbm.at[:, pl.ds(cs, width)],
                            slab_v.at[:, pl.ds(0, width)])

            def sweep(t, _):
                cols = gcol_v[pl.ds(t * _LANES, _LANES)]
                slot = t * _LANES + lanes
                m = jnp.logical_and(slot < gcount,
                                    jnp.logical_and(cols >= cs,
                                                    cols < cs + width))
                rloc = cols - cs
                for c in range(D):
                    x = dest_v[c, pl.ds(t * _LANES, _LANES)]
                    plsc.store_scatter(
                        slab_v,
                        [jnp.zeros((_LANES,), jnp.int32) + c, rloc],
                        x, mask=m)
                return 0

            lax.fori_loop(0, ngrp, sweep, 0)

            pltpu.sync_copy(slab_v.at[:, pl.ds(0, width)],
                            out_hbm.at[:, pl.ds(cs, width)])

        def chunk_step(ch, _):
            process_chunk(base + ch * _C, _C)
            return 0

        lax.fori_loop(0, nfull, chunk_step, 0)
        if rem:
            process_chunk(base + nfull * _C, rem)

        if extra_w:
            @pl.when(wid == NW - 1)
            def _():
                process_chunk(jnp.int32(extra_start), extra_w)

    return pl.kernel(
        body,
        out_type=jax.ShapeDtypeStruct((D, M), jnp.float32),
        mesh=mesh,
        scratch_types=[
            pltpu.VMEM((D, _C), jnp.float32),
            pltpu.VMEM((B,), jnp.int32),
            pltpu.VMEM((_GCAP + _LANES,), jnp.int32),
            pltpu.VMEM((_GCAP + _LANES,), jnp.int32),
            pltpu.VMEM((_PIECE, _VALW), jnp.float32),
            pltpu.VMEM((D, _GCAP), jnp.float32),
            pltpu.VMEM((D,), jnp.float32),
            pltpu.VMEM((D,), jnp.float32),
            pltpu.SemaphoreType.DMA,
        ],
        compiler_params=pltpu.CompilerParams(use_tc_tiling_on_sc=True,
                                             needs_layout_passes=False),
    )


def _swap_clamp(v, lower, upper):
    big = v >= upper[None, :]
    small = v <= lower[None, :]
    return jnp.where(big, lower[None, :], jnp.where(small, upper[None, :], v))


def kernel(mem, idx, val, lower, upper):
    M, D = mem.shape
    B = idx.shape[0]
    NC, NS = 2, 16

    valp = jnp.pad(val, ((0, 0), (0, _VALW - D)))
    fused = _make_fused_kernel(M, D, B, NC, NS)
    outt = fused(mem.T, idx, valp, lower, upper)

    maligned = M // 128 * 128
    ntail = M - maligned
    if ntail:
        cols = maligned + jnp.arange(ntail)
        eq = idx[:, None] == cols[None, :]
        winner = jnp.max(jnp.where(eq, jnp.arange(B)[:, None], -1), axis=0)
        wrow = _swap_clamp(val[jnp.maximum(winner, 0)], lower, upper)
        tail = jnp.where((winner >= 0)[None, :], wrow.T,
                         lax.dynamic_slice(mem.T, (0, maligned), (D, ntail)))
        outt = lax.dynamic_update_slice(outt, tail, (0, maligned))
    return outt.T


# speedup vs baseline: 1.5594x; 1.0061x over previous
import jax
import jax.numpy as jnp
from jax import lax
from jax.experimental import pallas as pl
from jax.experimental.pallas import tpu as pltpu
from jax.experimental.pallas import tpu_sc as plsc

_LANES = 16
_VALW = 128
_C = 1664
_GCAP = 768
_PIECE = 128


def _make_fused_kernel(M, D, B, NC, NS):
    NW = NC * NS
    maligned = M // 128 * 128
    wcols = maligned // NW // 128 * 128
    extra_start = wcols * NW
    extra_w = maligned - extra_start
    nfull = wcols // _C
    rem = wcols - nfull * _C
    mesh = plsc.VectorSubcoreMesh(core_axis_name="c", subcore_axis_name="s")

    def body(memt_hbm, idx_hbm, valp_hbm, lo_hbm, up_hbm, out_hbm,
             slab_v, idx_v, gcol_v, gb_v, stage_v, dest_v,
             lo_v, up_v, sem):
        wid = lax.axis_index("s") * NC + lax.axis_index("c")
        base = wid * wcols
        wend = jnp.where(wid == NW - 1, maligned, base + wcols)

        pltpu.sync_copy(idx_hbm, idx_v)
        pltpu.sync_copy(lo_hbm, lo_v)
        pltpu.sync_copy(up_hbm, up_v)

        lanes = lax.iota(jnp.int32, _LANES)
        zeros = jnp.zeros((_LANES,), jnp.int32)
        for z in range(_GCAP // _LANES):
            gb_v[pl.ds(z * _LANES, _LANES)] = zeros

        def popcount(m):
            return lax.reduce_max(plsc.all_reduce_population_count(m), (0,))

        def scan_step(t, gc):
            v = idx_v[pl.ds(t * _LANES, _LANES)]
            m = jnp.logical_and(v >= base, v < wend)
            plsc.store_compressed(gcol_v.at[pl.ds(gc, _LANES)], v, mask=m)
            plsc.store_compressed(gb_v.at[pl.ds(gc, _LANES)],
                                  t * _LANES + lanes, mask=m)
            return jnp.minimum(gc + popcount(m), _GCAP)

        gcount = lax.fori_loop(0, B // _LANES, scan_step, 0, unroll=8)
        ngrp = (gcount + _LANES - 1) // _LANES

        npiece = (gcount + _PIECE - 1) // _PIECE

        def fire(p):
            pltpu.async_copy(
                valp_hbm.at[gb_v.at[pl.ds(p * _PIECE, _PIECE)]],
                stage_v.at[pl.ds((p % 2) * _PIECE, _PIECE)], sem)

        fire(0)
        fire(1)

        def piece(p, _):
            pltpu.make_async_copy(
                valp_hbm.at[gb_v.at[pl.ds(p * _PIECE, _PIECE)]],
                stage_v.at[pl.ds((p % 2) * _PIECE, _PIECE)], sem).wait()
            rows = jnp.minimum(gcount - p * _PIECE, _PIECE)
            ngrp_p = (rows + _LANES - 1) // _LANES

            @plsc.parallel_loop(0, ngrp_p, unroll=2)
            def tp(j):
                ridx = (p % 2) * _PIECE + j * _LANES + lanes
                for c in range(D):
                    csplat = jnp.zeros((_LANES,), jnp.int32) + c
                    locs = plsc.load_gather(lo_v, [csplat])
                    upcs = plsc.load_gather(up_v, [csplat])
                    g = plsc.load_gather(stage_v, [ridx, csplat])
                    d = jnp.where(g >= upcs, locs,
                                  jnp.where(g <= locs, upcs, g))
                    dest_v[c, pl.ds(p * _PIECE + j * _LANES, _LANES)] = d

            @pl.when(p + 2 < npiece)
            def _():
                fire(p + 2)
            return 0

        lax.fori_loop(0, npiece, piece, 0)

        @pl.when(npiece < 2)
        def _():
            pltpu.make_async_copy(
                valp_hbm.at[gb_v.at[pl.ds(0, _PIECE)]],
                stage_v.at[pl.ds(_PIECE, _PIECE)], sem).wait()

        @pl.when(npiece < 1)
        def _():
            pltpu.make_async_copy(
                valp_hbm.at[gb_v.at[pl.ds(0, _PIECE)]],
                stage_v.at[pl.ds(0, _PIECE)], sem).wait()

        def process_chunk(cs, width):
            cs = pl.multiple_of(cs, 128)
            pltpu.sync_copy(memt_hbm.at[:, pl.ds(cs, width)],
                            slab_v.at[:, pl.ds(0, width)])

            def sweep(t, _):
                cols = gcol_v[pl.ds(t * _LANES, _LANES)]
                slot = t * _LANES + lanes
                m = jnp.logical_and(slot < gcount,
                                    jnp.logical_and(cols >= cs,
                                                    cols < cs + width))
                rloc = cols - cs
                for c in range(D):
                    x = dest_v[c, pl.ds(t * _LANES, _LANES)]
                    plsc.store_scatter(
                        slab_v,
                        [jnp.zeros((_LANES,), jnp.int32) + c, rloc],
                        x, mask=m)
                return 0

            lax.fori_loop(0, ngrp, sweep, 0)

            pltpu.sync_copy(slab_v.at[:, pl.ds(0, width)],
                            out_hbm.at[:, pl.ds(cs, width)])

        def chunk_step(ch, _):
            process_chunk(base + ch * _C, _C)
            return 0

        lax.fori_loop(0, nfull, chunk_step, 0)
        if rem:
            process_chunk(base + nfull * _C, rem)

        if extra_w:
            @pl.when(wid == NW - 1)
            def _():
                process_chunk(jnp.int32(extra_start), extra_w)

    return pl.kernel(
        body,
        out_type=jax.ShapeDtypeStruct((D, M), jnp.float32),
        mesh=mesh,
        scratch_types=[
            pltpu.VMEM((D, _C), jnp.float32),
            pltpu.VMEM((B,), jnp.int32),
            pltpu.VMEM((_GCAP + _LANES,), jnp.int32),
            pltpu.VMEM((_GCAP + _LANES,), jnp.int32),
            pltpu.VMEM((2 * _PIECE, _VALW), jnp.float32),
            pltpu.VMEM((D, _GCAP), jnp.float32),
            pltpu.VMEM((D,), jnp.float32),
            pltpu.VMEM((D,), jnp.float32),
            pltpu.SemaphoreType.DMA,
        ],
        compiler_params=pltpu.CompilerParams(use_tc_tiling_on_sc=True,
                                             needs_layout_passes=False),
    )


def _swap_clamp(v, lower, upper):
    big = v >= upper[None, :]
    small = v <= lower[None, :]
    return jnp.where(big, lower[None, :], jnp.where(small, upper[None, :], v))


def kernel(mem, idx, val, lower, upper):
    M, D = mem.shape
    B = idx.shape[0]
    NC, NS = 2, 16

    valp = jnp.pad(val, ((0, 0), (0, _VALW - D)))
    fused = _make_fused_kernel(M, D, B, NC, NS)
    outt = fused(mem.T, idx, valp, lower, upper)

    maligned = M // 128 * 128
    ntail = M - maligned
    if ntail:
        cols = maligned + jnp.arange(ntail)
        eq = idx[:, None] == cols[None, :]
        winner = jnp.max(jnp.where(eq, jnp.arange(B)[:, None], -1), axis=0)
        wrow = _swap_clamp(val[jnp.maximum(winner, 0)], lower, upper)
        tail = jnp.where((winner >= 0)[None, :], wrow.T,
                         lax.dynamic_slice(mem.T, (0, maligned), (D, ntail)))
        outt = lax.dynamic_update_slice(outt, tail, (0, maligned))
    return outt.T
